# baseline (speedup 1.0000x reference)
import jax
import jax.numpy as jnp
from jax.experimental import pallas as pl
from jax.experimental.pallas import tpu as pltpu

B, N, F, H, C = 128, 512, 256, 256, 10

CH = 4
NCH = B // CH
NBUF = 4


def _body(x_hbm, adj_hbm, w1_ref, b1_ref, w2_ref, b2_ref, out_hbm,
          xbuf, abuf, obuf, isem, osem):
    w1 = w1_ref[...].astype(jnp.bfloat16)
    w2 = w2_ref[...].astype(jnp.bfloat16)
    b1v = b1_ref[...]
    b2v = b2_ref[...]

    def in_copies(c, slot):
        return (
            pltpu.make_async_copy(x_hbm.at[pl.ds(c * CH, CH)],
                                  xbuf.at[slot], isem.at[slot, 0]),
            pltpu.make_async_copy(adj_hbm.at[pl.ds(c * CH, CH)],
                                  abuf.at[slot], isem.at[slot, 1]),
        )

    def out_copy(c, oslot):
        return pltpu.make_async_copy(obuf.at[oslot],
                                     out_hbm.at[pl.ds(c * CH, CH)],
                                     osem.at[oslot])

    for c in range(NBUF):
        for cp in in_copies(c, c % NBUF):
            cp.start()

    def step(c, carry):
        slot = jax.lax.rem(c, NBUF)
        oslot = jax.lax.rem(c, 2)
        for cp in in_copies(c, slot):
            cp.wait()

        @pl.when(c >= 2)
        def _():
            out_copy(c - 2, oslot).wait()

        for i in range(CH):
            a = abuf[slot, i].astype(jnp.bfloat16)
            h = jnp.dot(xbuf[slot, i].astype(jnp.bfloat16), w1,
                        preferred_element_type=jnp.float32)
            h = h + b1v
            h = jnp.dot(a, h.astype(jnp.bfloat16),
                        preferred_element_type=jnp.float32)
            h = jnp.maximum(h, 0.0)
            h = jnp.dot(h.astype(jnp.bfloat16), w2,
                        preferred_element_type=jnp.float32)
            h = h + b2v
            obuf[oslot, i] = jnp.dot(a, h.astype(jnp.bfloat16),
                                     preferred_element_type=jnp.float32)
        out_copy(c, oslot).start()

        @pl.when(c + NBUF < NCH)
        def _():
            for cp in in_copies(c + NBUF, slot):
                cp.start()

        return carry

    jax.lax.fori_loop(0, NCH, step, 0)
    for k in (NCH - 2, NCH - 1):
        out_copy(k, k % 2).wait()


def kernel(x, adj, W1, b1, W2, b2):
    b1r = b1.reshape(1, H)
    b2r = b2.reshape(1, C)
    hbm = pl.BlockSpec(memory_space=pltpu.MemorySpace.HBM)
    vmem = pl.BlockSpec(memory_space=pltpu.MemorySpace.VMEM)
    out = pl.pallas_call(
        _body,
        in_specs=[hbm, hbm, vmem, vmem, vmem, vmem],
        out_specs=hbm,
        out_shape=jax.ShapeDtypeStruct((B, N, C), jnp.float32),
        scratch_shapes=[
            pltpu.VMEM((NBUF, CH, N, F), jnp.float32),
            pltpu.VMEM((NBUF, CH, N, N), jnp.float32),
            pltpu.VMEM((2, CH, N, C), jnp.float32),
            pltpu.SemaphoreType.DMA((NBUF, 2)),
            pltpu.SemaphoreType.DMA((2,)),
        ],
    )(x, adj, W1, b1r, W2, b2r)
    return out[None]


# speedup vs baseline: 1.0720x; 1.0720x over previous
import jax
import jax.numpy as jnp
from jax.experimental import pallas as pl
from jax.experimental.pallas import tpu as pltpu

B, N, F, H, C = 128, 512, 256, 256, 10

CH = 8
NCH = B // CH
NBUF = 3


def _body(x_hbm, adj_hbm, w1_ref, b1_ref, w2_ref, b2_ref, out_hbm,
          xbuf, abuf, obuf, isem, osem):
    w1 = w1_ref[...].astype(jnp.bfloat16)
    w2 = w2_ref[...].astype(jnp.bfloat16)
    b1v = b1_ref[...]
    b2v = b2_ref[...]

    def in_copies(c, slot):
        return (
            pltpu.make_async_copy(x_hbm.at[pl.ds(c * CH, CH)],
                                  xbuf.at[slot], isem.at[slot, 0]),
            pltpu.make_async_copy(adj_hbm.at[pl.ds(c * CH, CH)],
                                  abuf.at[slot], isem.at[slot, 1]),
        )

    def out_copy(c, oslot):
        return pltpu.make_async_copy(obuf.at[oslot],
                                     out_hbm.at[pl.ds(c * CH, CH)],
                                     osem.at[oslot])

    for c in range(NBUF):
        for cp in in_copies(c, c % NBUF):
            cp.start()

    def step(c, carry):
        slot = jax.lax.rem(c, NBUF)
        oslot = jax.lax.rem(c, 2)
        for cp in in_copies(c, slot):
            cp.wait()

        @pl.when(c >= 2)
        def _():
            out_copy(c - 2, oslot).wait()

        for i in range(CH):
            a = abuf[slot, i].astype(jnp.bfloat16)
            h = jnp.dot(xbuf[slot, i].astype(jnp.bfloat16), w1,
                        preferred_element_type=jnp.float32)
            h = h + b1v
            h = jnp.dot(a, h.astype(jnp.bfloat16),
                        preferred_element_type=jnp.float32)
            h = jnp.maximum(h, 0.0)
            h = jnp.dot(h.astype(jnp.bfloat16), w2,
                        preferred_element_type=jnp.float32)
            h = h + b2v
            obuf[oslot, i] = jnp.dot(a, h.astype(jnp.bfloat16),
                                     preferred_element_type=jnp.float32)
        out_copy(c, oslot).start()

        @pl.when(c + NBUF < NCH)
        def _():
            for cp in in_copies(c + NBUF, slot):
                cp.start()

        return carry

    jax.lax.fori_loop(0, NCH, step, 0)
    for k in (NCH - 2, NCH - 1):
        out_copy(k, k % 2).wait()


def kernel(x, adj, W1, b1, W2, b2):
    b1r = b1.reshape(1, H)
    b2r = b2.reshape(1, C)
    hbm = pl.BlockSpec(memory_space=pltpu.MemorySpace.HBM)
    vmem = pl.BlockSpec(memory_space=pltpu.MemorySpace.VMEM)
    out = pl.pallas_call(
        _body,
        in_specs=[hbm, hbm, vmem, vmem, vmem, vmem],
        out_specs=hbm,
        out_shape=jax.ShapeDtypeStruct((B, N, C), jnp.float32),
        scratch_shapes=[
            pltpu.VMEM((NBUF, CH, N, F), jnp.float32),
            pltpu.VMEM((NBUF, CH, N, N), jnp.float32),
            pltpu.VMEM((2, CH, N, C), jnp.float32),
            pltpu.SemaphoreType.DMA((NBUF, 2)),
            pltpu.SemaphoreType.DMA((2,)),
        ],
    )(x, adj, W1, b1r, W2, b2r)
    return out[None]


# speedup vs baseline: 1.0745x; 1.0024x over previous
import jax
import jax.numpy as jnp
from jax.experimental import pallas as pl
from jax.experimental.pallas import tpu as pltpu

B, N, F, H, C = 128, 512, 256, 256, 10

CH = 8
NCH = B // CH
NBUF = 3
XSPLIT = 2
ASPLIT = 4


def _body(x_hbm, adj_hbm, w1_ref, b1_ref, w2_ref, b2_ref, out_hbm,
          xbuf, abuf, obuf, isem, osem):
    w1 = w1_ref[...].astype(jnp.bfloat16)
    w2 = w2_ref[...].astype(jnp.bfloat16)
    b1v = b1_ref[...]
    b2v = b2_ref[...]

    def in_copies(c, slot):
        cps = []
        for j in range(XSPLIT):
            w = CH // XSPLIT
            cps.append(pltpu.make_async_copy(
                x_hbm.at[pl.ds(c * CH + j * w, w)],
                xbuf.at[(slot, pl.ds(j * w, w))], isem.at[slot, j]))
        for j in range(ASPLIT):
            w = CH // ASPLIT
            cps.append(pltpu.make_async_copy(
                adj_hbm.at[pl.ds(c * CH + j * w, w)],
                abuf.at[(slot, pl.ds(j * w, w))], isem.at[slot, XSPLIT + j]))
        return cps

    def out_copy(c, oslot):
        return pltpu.make_async_copy(obuf.at[oslot],
                                     out_hbm.at[pl.ds(c * CH, CH)],
                                     osem.at[oslot])

    for c in range(NBUF):
        for cp in in_copies(c, c % NBUF):
            cp.start()

    def step(c, carry):
        slot = jax.lax.rem(c, NBUF)
        oslot = jax.lax.rem(c, 2)
        for cp in in_copies(c, slot):
            cp.wait()

        @pl.when(c >= 2)
        def _():
            out_copy(c - 2, oslot).wait()

        for i in range(CH):
            a = abuf[slot, i].astype(jnp.bfloat16)
            h = jnp.dot(xbuf[slot, i].astype(jnp.bfloat16), w1,
                        preferred_element_type=jnp.float32)
            h = h + b1v
            h = jnp.dot(a, h.astype(jnp.bfloat16),
                        preferred_element_type=jnp.float32)
            h = jnp.maximum(h, 0.0)
            h = jnp.dot(h.astype(jnp.bfloat16), w2,
                        preferred_element_type=jnp.float32)
            h = h + b2v
            obuf[oslot, i] = jnp.dot(a, h.astype(jnp.bfloat16),
                                     preferred_element_type=jnp.float32)
        out_copy(c, oslot).start()

        @pl.when(c + NBUF < NCH)
        def _():
            for cp in in_copies(c + NBUF, slot):
                cp.start()

        return carry

    jax.lax.fori_loop(0, NCH, step, 0)
    for k in (NCH - 2, NCH - 1):
        out_copy(k, k % 2).wait()


def kernel(x, adj, W1, b1, W2, b2):
    b1r = b1.reshape(1, H)
    b2r = b2.reshape(1, C)
    hbm = pl.BlockSpec(memory_space=pltpu.MemorySpace.HBM)
    vmem = pl.BlockSpec(memory_space=pltpu.MemorySpace.VMEM)
    out = pl.pallas_call(
        _body,
        in_specs=[hbm, hbm, vmem, vmem, vmem, vmem],
        out_specs=hbm,
        out_shape=jax.ShapeDtypeStruct((B, N, C), jnp.float32),
        scratch_shapes=[
            pltpu.VMEM((NBUF, CH, N, F), jnp.float32),
            pltpu.VMEM((NBUF, CH, N, N), jnp.float32),
            pltpu.VMEM((2, CH, N, C), jnp.float32),
            pltpu.SemaphoreType.DMA((NBUF, XSPLIT + ASPLIT)),
            pltpu.SemaphoreType.DMA((2,)),
        ],
    )(x, adj, W1, b1r, W2, b2r)
    return out[None]
